# baseline (speedup 1.0000x reference)
import functools

import jax
import jax.numpy as jnp
from jax import lax
from jax.experimental import pallas as pl
from jax.experimental.pallas import tpu as pltpu
from jax.experimental.pallas import tpu_sc as plsc

EPS = 1e-5


def _norm_body(tab_ref, g_ref, b_ref, out_ref):
    x = tab_ref[...]
    mean = jnp.mean(x, axis=1, keepdims=True)
    xc = x - mean
    var = jnp.mean(xc * xc, axis=1, keepdims=True)
    inv = lax.rsqrt(var + EPS)
    out_ref[...] = xc * inv * g_ref[...] + b_ref[...]


def _normalize_table(table, gamma, beta):
    V, D = table.shape
    R = 2048
    grid = (V + R - 1) // R
    return pl.pallas_call(
        _norm_body,
        grid=(grid,),
        in_specs=[
            pl.BlockSpec((R, D), lambda i: (i, 0)),
            pl.BlockSpec((1, D), lambda i: (0, 0)),
            pl.BlockSpec((1, D), lambda i: (0, 0)),
        ],
        out_specs=pl.BlockSpec((R, D), lambda i: (i, 0)),
        out_shape=jax.ShapeDtypeStruct((V, D), jnp.float32),
    )(table, gamma.reshape(1, D), beta.reshape(1, D))


_HP = 8
_SB = 128


@functools.partial(jax.jit, static_argnums=(2,))
def _sc_gather_t(ntable, ids_t, d):
    info = plsc.get_sparse_core_info()
    nc, ns = info.num_cores, info.num_subcores
    nw = nc * ns
    h, b = ids_t.shape
    nbj = b // _SB
    nhr = h // _HP
    nitems = (nbj * nhr) // nw
    dblks = d // 8

    mesh = plsc.VectorSubcoreMesh(core_axis_name="c", subcore_axis_name="s")

    @functools.partial(
        pl.kernel,
        mesh=mesh,
        out_type=jax.ShapeDtypeStruct((h, dblks, nbj, 8, _SB), jnp.float32),
        scratch_types=[
            pltpu.VMEM((_HP, _SB), jnp.int32),
            pltpu.VMEM((_HP, _SB), jnp.int32),
            pltpu.VMEM((_HP * _SB, d), jnp.float32),
            pltpu.VMEM((_HP * _SB, d), jnp.float32),
            pltpu.VMEM((_HP, dblks, 1, 8, _SB + 1), jnp.float32),
            pltpu.VMEM((_HP, dblks, 1, 8, _SB + 1), jnp.float32),
        ]
        + [pltpu.SemaphoreType.DMA] * 4,
        compiler_params=pltpu.CompilerParams(
            use_tc_tiling_on_sc=False, needs_layout_passes=False),
    )
    def k(tab_hbm, ids_hbm, out_hbm, i0, i1, g0, g1, t0, t1, gs0, gs1, os0, os1):
        idxs, gbufs, tbufs = (i0, i1), (g0, g1), (t0, t1)
        gsems, osems = (gs0, gs1), (os0, os1)
        wid = lax.axis_index("s") * nc + lax.axis_index("c")
        gbase = wid * nitems
        iota = lax.iota(jnp.int32, 16)
        p_dblk = iota // 8
        p_dsub = iota % 8
        zv = jnp.zeros((16,), jnp.int32)

        def coords(i):
            g = gbase + i
            return g // nhr, g % nhr

        def fire(i, bb):
            bj, hr = coords(i)
            pltpu.sync_copy(
                ids_hbm.at[pl.ds(hr * _HP, _HP), pl.ds(bj * _SB, _SB)],
                idxs[bb])
            for s in range(_HP):
                pltpu.async_copy(
                    tab_hbm.at[idxs[bb].at[s]],
                    gbufs[bb].at[pl.ds(s * _SB, _SB)],
                    gsems[bb])

        def gather_wait(bb):
            pltpu.make_async_copy(
                tab_hbm.at[pl.ds(0, _HP * _SB)], gbufs[bb], gsems[bb]).wait()

        def store_cps(i, bb):
            bj, hr = coords(i)
            return [
                pltpu.make_async_copy(
                    tbufs[bb].at[:, pl.ds(dblk, 1), :, :, pl.ds(0, _SB)],
                    out_hbm.at[pl.ds(hr * _HP, _HP), pl.ds(dblk, 1),
                               pl.ds(bj, 1), :, :],
                    osems[bb])
                for dblk in range(dblks)
            ]

        def transpose(bb):
            gb, tb = gbufs[bb], tbufs[bb]

            def hbody(h_, carry):
                h16 = zv + h_
                for bl in range(_SB):
                    v = gb[h_ * _SB + bl, :]
                    plsc.store_scatter(
                        tb, [h16, p_dblk, zv, p_dsub, zv + bl], v)
                return carry

            lax.fori_loop(0, _HP, hbody, 0)

        fire(0, 0)

        def round_body(r, carry):
            for bb in range(2):
                i = 2 * r + bb
                gather_wait(bb)

                @pl.when(i + 1 < nitems)
                def _():
                    @pl.when(i >= 1)
                    def _():
                        for cp in store_cps(i - 1, 1 - bb):
                            cp.wait()

                    fire(i + 1, 1 - bb)

                transpose(bb)
                for cp in store_cps(i, bb):
                    cp.start()
            return carry

        lax.fori_loop(0, nitems // 2, round_body, 0)
        for cp in store_cps(nitems - 2, 0):
            cp.wait()
        for cp in store_cps(nitems - 1, 1):
            cp.wait()

    return k(ntable, ids_t)


def kernel(venue_id, table, gamma, beta):
    B, H = venue_id.shape
    V, D = table.shape
    ntable = _normalize_table(table.astype(jnp.float32), gamma, beta)
    ids_t = venue_id.astype(jnp.int32).T
    out5 = _sc_gather_t(ntable, ids_t, D)
    return jnp.transpose(out5, (2, 4, 0, 1, 3)).reshape(B, H, D)


# speedup vs baseline: 1.1795x; 1.1795x over previous
import functools

import jax
import jax.numpy as jnp
from jax import lax
from jax.experimental import pallas as pl
from jax.experimental.pallas import tpu as pltpu
from jax.experimental.pallas import tpu_sc as plsc

EPS = 1e-5


def _norm_body(tab_ref, g_ref, b_ref, out_ref):
    x = tab_ref[...]
    mean = jnp.mean(x, axis=1, keepdims=True)
    xc = x - mean
    var = jnp.mean(xc * xc, axis=1, keepdims=True)
    inv = lax.rsqrt(var + EPS)
    out_ref[...] = xc * inv * g_ref[...] + b_ref[...]


def _normalize_table(table, gamma, beta):
    V, D = table.shape
    R = 2048
    grid = (V + R - 1) // R
    return pl.pallas_call(
        _norm_body,
        grid=(grid,),
        in_specs=[
            pl.BlockSpec((R, D), lambda i: (i, 0)),
            pl.BlockSpec((1, D), lambda i: (0, 0)),
            pl.BlockSpec((1, D), lambda i: (0, 0)),
        ],
        out_specs=pl.BlockSpec((R, D), lambda i: (i, 0)),
        out_shape=jax.ShapeDtypeStruct((V, D), jnp.float32),
    )(table, gamma.reshape(1, D), beta.reshape(1, D))


_HP = 8
_SB = 128


@functools.partial(jax.jit, static_argnums=(2,))
def _sc_gather_t(ntable, ids_t, d):
    info = plsc.get_sparse_core_info()
    nc, ns = info.num_cores, info.num_subcores
    nw = nc * ns
    h, b = ids_t.shape
    nbj = b // _SB
    nhr = h // _HP
    nitems = (nbj * nhr) // nw
    dblks = d // 8

    mesh = plsc.VectorSubcoreMesh(core_axis_name="c", subcore_axis_name="s")

    @functools.partial(
        pl.kernel,
        mesh=mesh,
        out_type=jax.ShapeDtypeStruct((h, dblks, nbj, 8, _SB), jnp.float32),
        scratch_types=[
            pltpu.VMEM((_HP, _SB), jnp.int32),
            pltpu.VMEM((_HP, _SB), jnp.int32),
            pltpu.VMEM((_HP * _SB, d), jnp.float32),
            pltpu.VMEM((_HP * _SB, d), jnp.float32),
            pltpu.VMEM((_HP, dblks, 1, 8, _SB + 1), jnp.float32),
            pltpu.VMEM((_HP, dblks, 1, 8, _SB + 1), jnp.float32),
        ]
        + [pltpu.SemaphoreType.DMA] * 4,
        compiler_params=pltpu.CompilerParams(
            use_tc_tiling_on_sc=False, needs_layout_passes=False),
    )
    def k(tab_hbm, ids_hbm, out_hbm, i0, i1, g0, g1, t0, t1, gs0, gs1, os0, os1):
        idxs, gbufs, tbufs = (i0, i1), (g0, g1), (t0, t1)
        gsems, osems = (gs0, gs1), (os0, os1)
        wid = lax.axis_index("s") * nc + lax.axis_index("c")
        gbase = wid * nitems
        iota = lax.iota(jnp.int32, 16)
        p_dblk = iota // 8
        p_dsub = iota % 8
        zv = jnp.zeros((16,), jnp.int32)

        def coords(i):
            g = gbase + i
            return g // nhr, g % nhr

        def fire(i, bb):
            bj, hr = coords(i)
            pltpu.sync_copy(
                ids_hbm.at[pl.ds(hr * _HP, _HP), pl.ds(bj * _SB, _SB)],
                idxs[bb])
            for s in range(_HP):
                pltpu.async_copy(
                    tab_hbm.at[idxs[bb].at[s]],
                    gbufs[bb].at[pl.ds(s * _SB, _SB)],
                    gsems[bb])

        def gather_wait(bb):
            pltpu.make_async_copy(
                tab_hbm.at[pl.ds(0, _HP * _SB)], gbufs[bb], gsems[bb]).wait()

        def store_cps(i, bb):
            bj, hr = coords(i)
            return [
                pltpu.make_async_copy(
                    tbufs[bb].at[:, pl.ds(dblk, 1), :, :, pl.ds(0, _SB)],
                    out_hbm.at[pl.ds(hr * _HP, _HP), pl.ds(dblk, 1),
                               pl.ds(bj, 1), :, :],
                    osems[bb])
                for dblk in range(dblks)
            ]

        def transpose(bb):
            gb, tb = gbufs[bb], tbufs[bb]

            def hbody(h_, carry):
                h16 = zv + h_

                def blbody(bq, carry2):
                    for j in range(8):
                        bl = bq * 8 + j
                        v = gb[h_ * _SB + bl, :]
                        plsc.store_scatter(
                            tb, [h16, p_dblk, zv, p_dsub, zv + bl], v)
                    return carry2

                lax.fori_loop(0, _SB // 8, blbody, carry)
                return carry

            lax.fori_loop(0, _HP, hbody, 0)

        fire(0, 0)

        def round_body(r, carry):
            for bb in range(2):
                i = 2 * r + bb
                gather_wait(bb)

                @pl.when(i + 1 < nitems)
                def _():
                    @pl.when(i >= 1)
                    def _():
                        for cp in store_cps(i - 1, 1 - bb):
                            cp.wait()

                    fire(i + 1, 1 - bb)

                transpose(bb)
                for cp in store_cps(i, bb):
                    cp.start()
            return carry

        lax.fori_loop(0, nitems // 2, round_body, 0)
        for cp in store_cps(nitems - 2, 0):
            cp.wait()
        for cp in store_cps(nitems - 1, 1):
            cp.wait()

    return k(ntable, ids_t)


def kernel(venue_id, table, gamma, beta):
    B, H = venue_id.shape
    V, D = table.shape
    ntable = _normalize_table(table.astype(jnp.float32), gamma, beta)
    ids_t = venue_id.astype(jnp.int32).T
    out5 = _sc_gather_t(ntable, ids_t, D)
    return jnp.transpose(out5, (2, 4, 0, 1, 3)).reshape(B, H, D)


# speedup vs baseline: 1.3428x; 1.1384x over previous
import functools

import jax
import jax.numpy as jnp
from jax import lax
from jax.experimental import pallas as pl
from jax.experimental.pallas import tpu as pltpu
from jax.experimental.pallas import tpu_sc as plsc

EPS = 1e-5


def _norm_body(tab_ref, g_ref, b_ref, out_ref):
    x = tab_ref[...]
    mean = jnp.mean(x, axis=1, keepdims=True)
    xc = x - mean
    var = jnp.mean(xc * xc, axis=1, keepdims=True)
    inv = lax.rsqrt(var + EPS)
    out_ref[...] = xc * inv * g_ref[...] + b_ref[...]


def _normalize_table(table, gamma, beta):
    V, D = table.shape
    R = 2048
    grid = (V + R - 1) // R
    return pl.pallas_call(
        _norm_body,
        grid=(grid,),
        in_specs=[
            pl.BlockSpec((R, D), lambda i: (i, 0)),
            pl.BlockSpec((1, D), lambda i: (0, 0)),
            pl.BlockSpec((1, D), lambda i: (0, 0)),
        ],
        out_specs=pl.BlockSpec((R, D), lambda i: (i, 0)),
        out_shape=jax.ShapeDtypeStruct((V, D), jnp.float32),
    )(table, gamma.reshape(1, D), beta.reshape(1, D))


_HP = 8
_SB = 128


@functools.partial(jax.jit, static_argnums=(2,))
def _sc_gather_t(ntable, ids_t, d):
    info = plsc.get_sparse_core_info()
    nc, ns = info.num_cores, info.num_subcores
    nw = nc * ns
    h, b = ids_t.shape
    nbj = b // _SB
    nhr = h // _HP
    nitems = (nbj * nhr) // nw
    dblks = d // 8

    mesh = plsc.VectorSubcoreMesh(core_axis_name="c", subcore_axis_name="s")

    @functools.partial(
        pl.kernel,
        mesh=mesh,
        out_type=jax.ShapeDtypeStruct((h, dblks, nbj, 8, _SB), jnp.float32),
        scratch_types=[
            pltpu.VMEM((_HP, _SB), jnp.int32),
            pltpu.VMEM((_HP, _SB), jnp.int32),
            pltpu.VMEM((_HP * _SB, d), jnp.float32),
            pltpu.VMEM((_HP * _SB, d), jnp.float32),
            pltpu.VMEM((_HP, dblks, 1, 8, _SB + 1), jnp.float32),
            pltpu.VMEM((_HP, dblks, 1, 8, _SB + 1), jnp.float32),
        ]
        + [pltpu.SemaphoreType.DMA] * 6,
        compiler_params=pltpu.CompilerParams(
            use_tc_tiling_on_sc=False, needs_layout_passes=False),
    )
    def k(tab_hbm, ids_hbm, out_hbm, i0, i1, g0, g1, t0, t1,
          gs0, gs1, os0, os1, is0, is1):
        idxs, gbufs, tbufs = (i0, i1), (g0, g1), (t0, t1)
        gsems, osems, isems = (gs0, gs1), (os0, os1), (is0, is1)
        wid = lax.axis_index("s") * nc + lax.axis_index("c")
        gbase = wid * nitems
        iota = lax.iota(jnp.int32, 16)
        p_dblk = iota // 8
        p_dsub = iota % 8
        zv = jnp.zeros((16,), jnp.int32)

        def coords(i):
            g = gbase + i
            return g // nhr, g % nhr

        def idx_cp(i, bb):
            bj, hr = coords(i)
            return pltpu.make_async_copy(
                ids_hbm.at[pl.ds(hr * _HP, _HP), pl.ds(bj * _SB, _SB)],
                idxs[bb], isems[bb])

        def fire(i, bb):
            idx_cp(i, bb).wait()
            for s in range(_HP):
                pltpu.async_copy(
                    tab_hbm.at[idxs[bb].at[s]],
                    gbufs[bb].at[pl.ds(s * _SB, _SB)],
                    gsems[bb])

        def gather_wait(bb):
            pltpu.make_async_copy(
                tab_hbm.at[pl.ds(0, _HP * _SB)], gbufs[bb], gsems[bb]).wait()

        def store_cps(i, bb):
            bj, hr = coords(i)
            return [
                pltpu.make_async_copy(
                    tbufs[bb].at[:, pl.ds(dblk, 1), :, :, pl.ds(0, _SB)],
                    out_hbm.at[pl.ds(hr * _HP, _HP), pl.ds(dblk, 1),
                               pl.ds(bj, 1), :, :],
                    osems[bb])
                for dblk in range(dblks)
            ]

        def transpose(bb):
            gb, tb = gbufs[bb], tbufs[bb]

            def hbody(h_, carry):
                h16 = zv + h_

                def blbody(bq, carry2):
                    for j in range(8):
                        bl = bq * 8 + j
                        v = gb[h_ * _SB + bl, :]
                        plsc.store_scatter(
                            tb, [h16, p_dblk, zv, p_dsub, zv + bl], v)
                    return carry2

                lax.fori_loop(0, _SB // 8, blbody, carry)
                return carry

            lax.fori_loop(0, _HP, hbody, 0)

        idx_cp(0, 0).start()
        fire(0, 0)
        idx_cp(1, 1).start()

        def round_body(r, carry):
            for bb in range(2):
                i = 2 * r + bb
                gather_wait(bb)

                @pl.when(i + 2 < nitems)
                def _():
                    idx_cp(i + 2, bb).start()

                @pl.when(i + 1 < nitems)
                def _():
                    @pl.when(i >= 1)
                    def _():
                        for cp in store_cps(i - 1, 1 - bb):
                            cp.wait()

                    fire(i + 1, 1 - bb)

                transpose(bb)
                for cp in store_cps(i, bb):
                    cp.start()
            return carry

        lax.fori_loop(0, nitems // 2, round_body, 0)
        for cp in store_cps(nitems - 2, 0):
            cp.wait()
        for cp in store_cps(nitems - 1, 1):
            cp.wait()

    return k(ntable, ids_t)


def kernel(venue_id, table, gamma, beta):
    B, H = venue_id.shape
    V, D = table.shape
    ntable = _normalize_table(table.astype(jnp.float32), gamma, beta)
    ids_t = venue_id.astype(jnp.int32).T
    out5 = _sc_gather_t(ntable, ids_t, D)
    return jnp.transpose(out5, (2, 4, 0, 1, 3)).reshape(B, H, D)


# speedup vs baseline: 1.4573x; 1.0853x over previous
import functools

import jax
import jax.numpy as jnp
from jax import lax
from jax.experimental import pallas as pl
from jax.experimental.pallas import tpu as pltpu
from jax.experimental.pallas import tpu_sc as plsc

EPS = 1e-5


def _norm_body(tab_ref, g_ref, b_ref, out_ref):
    x = tab_ref[...]
    mean = jnp.mean(x, axis=0, keepdims=True)
    xc = x - mean
    var = jnp.mean(xc * xc, axis=0, keepdims=True)
    inv = lax.rsqrt(var + EPS)
    out_ref[...] = (xc * inv * g_ref[...] + b_ref[...]).T


def _normalize_table(table_t, gamma, beta):
    D, V = table_t.shape
    R = 2048
    grid = (V + R - 1) // R
    return pl.pallas_call(
        _norm_body,
        grid=(grid,),
        in_specs=[
            pl.BlockSpec((D, R), lambda i: (0, i)),
            pl.BlockSpec((D, 1), lambda i: (0, 0)),
            pl.BlockSpec((D, 1), lambda i: (0, 0)),
        ],
        out_specs=pl.BlockSpec((R, D), lambda i: (i, 0)),
        out_shape=jax.ShapeDtypeStruct((V, D), jnp.float32),
    )(table_t, gamma.reshape(D, 1), beta.reshape(D, 1))


_HP = 8
_SB = 128


@functools.partial(jax.jit, static_argnums=(2,))
def _sc_gather_t(ntable, ids_t, d):
    info = plsc.get_sparse_core_info()
    nc, ns = info.num_cores, info.num_subcores
    nw = nc * ns
    h, b = ids_t.shape
    nbj = b // _SB
    nhr = h // _HP
    nitems = (nbj * nhr) // nw
    dblks = d // 8

    mesh = plsc.VectorSubcoreMesh(core_axis_name="c", subcore_axis_name="s")

    @functools.partial(
        pl.kernel,
        mesh=mesh,
        out_type=jax.ShapeDtypeStruct((h, dblks, nbj, 8, _SB), jnp.float32),
        scratch_types=[
            pltpu.VMEM((_HP, _SB), jnp.int32),
            pltpu.VMEM((_HP, _SB), jnp.int32),
            pltpu.VMEM((_HP * _SB, d), jnp.float32),
            pltpu.VMEM((_HP * _SB, d), jnp.float32),
            pltpu.VMEM((_HP, dblks, 1, 8, _SB + 1), jnp.float32),
            pltpu.VMEM((_HP, dblks, 1, 8, _SB + 1), jnp.float32),
        ]
        + [pltpu.SemaphoreType.DMA] * 6,
        compiler_params=pltpu.CompilerParams(
            use_tc_tiling_on_sc=False, needs_layout_passes=False),
    )
    def k(tab_hbm, ids_hbm, out_hbm, i0, i1, g0, g1, t0, t1,
          gs0, gs1, os0, os1, is0, is1):
        idxs, gbufs, tbufs = (i0, i1), (g0, g1), (t0, t1)
        gsems, osems, isems = (gs0, gs1), (os0, os1), (is0, is1)
        wid = lax.axis_index("s") * nc + lax.axis_index("c")
        gbase = wid * nitems
        iota = lax.iota(jnp.int32, 16)
        p_dblk = iota // 8
        p_dsub = iota % 8
        zv = jnp.zeros((16,), jnp.int32)

        def coords(i):
            g = gbase + i
            return g // nhr, g % nhr

        def idx_cp(i, bb):
            bj, hr = coords(i)
            return pltpu.make_async_copy(
                ids_hbm.at[pl.ds(hr * _HP, _HP), pl.ds(bj * _SB, _SB)],
                idxs[bb], isems[bb])

        def fire(i, bb):
            idx_cp(i, bb).wait()
            for s in range(_HP):
                pltpu.async_copy(
                    tab_hbm.at[idxs[bb].at[s]],
                    gbufs[bb].at[pl.ds(s * _SB, _SB)],
                    gsems[bb])

        def gather_wait(bb):
            pltpu.make_async_copy(
                tab_hbm.at[pl.ds(0, _HP * _SB)], gbufs[bb], gsems[bb]).wait()

        def store_cps(i, bb):
            bj, hr = coords(i)
            return [
                pltpu.make_async_copy(
                    tbufs[bb].at[:, pl.ds(dblk, 1), :, :, pl.ds(0, _SB)],
                    out_hbm.at[pl.ds(hr * _HP, _HP), pl.ds(dblk, 1),
                               pl.ds(bj, 1), :, :],
                    osems[bb])
                for dblk in range(dblks)
            ]

        def transpose(bb):
            gb, tb = gbufs[bb], tbufs[bb]

            def hbody(h_, carry):
                h16 = zv + h_

                def blbody(bq, carry2):
                    for j in range(8):
                        bl = bq * 8 + j
                        v = gb[h_ * _SB + bl, :]
                        plsc.store_scatter(
                            tb, [h16, p_dblk, zv, p_dsub, zv + bl], v)
                    return carry2

                lax.fori_loop(0, _SB // 8, blbody, carry)
                return carry

            lax.fori_loop(0, _HP, hbody, 0)

        idx_cp(0, 0).start()
        fire(0, 0)
        idx_cp(1, 1).start()

        def round_body(r, carry):
            for bb in range(2):
                i = 2 * r + bb
                gather_wait(bb)

                @pl.when(i + 2 < nitems)
                def _():
                    idx_cp(i + 2, bb).start()

                @pl.when(i + 1 < nitems)
                def _():
                    @pl.when(i >= 1)
                    def _():
                        for cp in store_cps(i - 1, 1 - bb):
                            cp.wait()

                    fire(i + 1, 1 - bb)

                transpose(bb)
                for cp in store_cps(i, bb):
                    cp.start()
            return carry

        lax.fori_loop(0, nitems // 2, round_body, 0)
        for cp in store_cps(nitems - 2, 0):
            cp.wait()
        for cp in store_cps(nitems - 1, 1):
            cp.wait()

    return k(ntable, ids_t)


def kernel(venue_id, table, gamma, beta):
    B, H = venue_id.shape
    V, D = table.shape
    ntable = _normalize_table(table.astype(jnp.float32).T, gamma, beta)
    ids_t = venue_id.astype(jnp.int32).T
    out5 = _sc_gather_t(ntable, ids_t, D)
    return jnp.transpose(out5, (2, 4, 0, 1, 3)).reshape(B, H, D)


# speedup vs baseline: 1.5133x; 1.0385x over previous
import functools

import jax
import jax.numpy as jnp
from jax import lax
from jax.experimental import pallas as pl
from jax.experimental.pallas import tpu as pltpu
from jax.experimental.pallas import tpu_sc as plsc

EPS = 1e-5


def _norm_body(tab_ref, g_ref, b_ref, out_ref):
    x = tab_ref[...]
    mean = jnp.mean(x, axis=0, keepdims=True)
    xc = x - mean
    var = jnp.mean(xc * xc, axis=0, keepdims=True)
    inv = lax.rsqrt(var + EPS)
    out_ref[...] = (xc * inv * g_ref[...] + b_ref[...]).T


def _normalize_table(table_t, gamma, beta):
    D, V = table_t.shape
    R = 2048
    grid = (V + R - 1) // R
    return pl.pallas_call(
        _norm_body,
        grid=(grid,),
        in_specs=[
            pl.BlockSpec((D, R), lambda i: (0, i)),
            pl.BlockSpec((D, 1), lambda i: (0, 0)),
            pl.BlockSpec((D, 1), lambda i: (0, 0)),
        ],
        out_specs=pl.BlockSpec((R, D), lambda i: (i, 0)),
        out_shape=jax.ShapeDtypeStruct((V, D), jnp.float32),
    )(table_t, gamma.reshape(D, 1), beta.reshape(D, 1))


_HP = 8
_SB = 128


@functools.partial(jax.jit, static_argnums=(2,))
def _sc_gather_t(ntable, ids_t, d):
    info = plsc.get_sparse_core_info()
    nc, ns = info.num_cores, info.num_subcores
    nw = nc * ns
    h, b = ids_t.shape
    nbj = b // _SB
    nhr = h // _HP
    nitems = (nbj * nhr) // nw
    dblks = d // 8

    mesh = plsc.VectorSubcoreMesh(core_axis_name="c", subcore_axis_name="s")

    @functools.partial(
        pl.kernel,
        mesh=mesh,
        out_type=jax.ShapeDtypeStruct((h, dblks, nbj, 8, _SB), jnp.float32),
        scratch_types=[pltpu.VMEM((_HP, _SB), jnp.int32)] * 3
        + [pltpu.VMEM((_HP * _SB, d), jnp.float32)] * 3
        + [pltpu.VMEM((_HP, dblks, 1, 8, _SB + 1), jnp.float32)] * 3
        + [pltpu.SemaphoreType.DMA] * 9,
        compiler_params=pltpu.CompilerParams(
            use_tc_tiling_on_sc=False, needs_layout_passes=False),
    )
    def k(tab_hbm, ids_hbm, out_hbm, i0, i1, i2, g0, g1, g2, t0, t1, t2,
          gs0, gs1, gs2, os0, os1, os2, is0, is1, is2):
        idxs, gbufs, tbufs = (i0, i1, i2), (g0, g1, g2), (t0, t1, t2)
        gsems, osems, isems = (gs0, gs1, gs2), (os0, os1, os2), (is0, is1, is2)
        wid = lax.axis_index("s") * nc + lax.axis_index("c")
        gbase = wid * nitems
        iota = lax.iota(jnp.int32, 16)
        p_dblk = iota // 8
        p_dsub = iota % 8
        zv = jnp.zeros((16,), jnp.int32)

        def coords(i):
            g = gbase + i
            return g // nhr, g % nhr

        def idx_cp(i, bb):
            bj, hr = coords(i)
            return pltpu.make_async_copy(
                ids_hbm.at[pl.ds(hr * _HP, _HP), pl.ds(bj * _SB, _SB)],
                idxs[bb], isems[bb])

        def fire(i, bb):
            idx_cp(i, bb).wait()
            for s in range(_HP):
                pltpu.async_copy(
                    tab_hbm.at[idxs[bb].at[s]],
                    gbufs[bb].at[pl.ds(s * _SB, _SB)],
                    gsems[bb])

        def gather_wait(bb):
            pltpu.make_async_copy(
                tab_hbm.at[pl.ds(0, _HP * _SB)], gbufs[bb], gsems[bb]).wait()

        def store_cps(i, bb):
            bj, hr = coords(i)
            return [
                pltpu.make_async_copy(
                    tbufs[bb].at[:, pl.ds(dblk, 1), :, :, pl.ds(0, _SB)],
                    out_hbm.at[pl.ds(hr * _HP, _HP), pl.ds(dblk, 1),
                               pl.ds(bj, 1), :, :],
                    osems[bb])
                for dblk in range(dblks)
            ]

        def transpose(bb):
            gb, tb = gbufs[bb], tbufs[bb]

            def hbody(h_, carry):
                h16 = zv + h_

                def blbody(bq, carry2):
                    for j in range(8):
                        bl = bq * 8 + j
                        v = gb[h_ * _SB + bl, :]
                        plsc.store_scatter(
                            tb, [h16, p_dblk, zv, p_dsub, zv + bl], v)
                    return carry2

                lax.fori_loop(0, _SB // 8, blbody, carry)
                return carry

            lax.fori_loop(0, _HP, hbody, 0)

        idx_cp(0, 0).start()
        idx_cp(1, 1).start()
        idx_cp(2, 2).start()
        fire(0, 0)
        fire(1, 1)

        nrounds = (nitems + 2) // 3

        def round_body(r, carry):
            for bb in range(3):
                i = 3 * r + bb

                b2 = (bb + 2) % 3

                @pl.when(i < nitems)
                def _():
                    gather_wait(bb)

                    @pl.when(i + 3 < nitems)
                    def _():
                        idx_cp(i + 3, bb).start()

                    @pl.when(i + 2 < nitems)
                    def _():
                        fire(i + 2, b2)

                    @pl.when(i >= 3)
                    def _():
                        for cp in store_cps(i - 3, bb):
                            cp.wait()

                    transpose(bb)
                    for cp in store_cps(i, bb):
                        cp.start()

            return carry

        lax.fori_loop(0, nrounds, round_body, 0)
        for j in range(3):
            for cp in store_cps(nitems - 3 + j, (nitems - 3 + j) % 3):
                cp.wait()

    return k(ntable, ids_t)


def kernel(venue_id, table, gamma, beta):
    B, H = venue_id.shape
    V, D = table.shape
    ntable = _normalize_table(table.astype(jnp.float32).T, gamma, beta)
    ids_t = venue_id.astype(jnp.int32).T
    out5 = _sc_gather_t(ntable, ids_t, D)
    return jnp.transpose(out5, (2, 4, 0, 1, 3)).reshape(B, H, D)
